# baseline (speedup 1.0000x reference)
import functools

import jax
import jax.numpy as jnp
from jax import lax
from jax.experimental import pallas as pl
from jax.experimental.pallas import tpu as pltpu
from jax.experimental.pallas import tpu_sc as plsc

SIZE = 64
CAPACITY = 1000000

_info = plsc.get_sparse_core_info()
NC = _info.num_cores
NS = _info.num_subcores
NW = NC * NS
L = 16

CHUNK = 256
W_KEYS = 4864
NCH = W_KEYS // CHUNK
GROUPS = CHUNK // L
SC_END = W_KEYS * NW
TC_BLOCK = 512
TC_END = 999936
TC_NBLK = (TC_END - SC_END) // TC_BLOCK

_NEG = -3.4e38
_EPS2 = 1e-16


def _perm(v, idx):
    dnums = lax.GatherDimensionNumbers(
        offset_dims=(), collapsed_slice_dims=(0,), start_index_map=(0,))
    return lax.gather(v, idx[:, None], dnums, (1,),
                      mode=lax.GatherScatterMode.PROMISE_IN_BOUNDS)


def _tree(v, io, op):
    for sh in (8, 4, 2, 1):
        v = op(v, _perm(v, io ^ sh))
    return v


def _make_sc_scan():
    mesh = plsc.VectorSubcoreMesh(core_axis_name="c", subcore_axis_name="s")

    @functools.partial(
        pl.kernel,
        out_type=[
            jax.ShapeDtypeStruct((NW, L), jnp.float32),
            jax.ShapeDtypeStruct((NW, L), jnp.int32),
        ],
        mesh=mesh,
        compiler_params=pltpu.CompilerParams(use_tc_tiling_on_sc=True),
        scratch_types=[
            pltpu.VMEM((1, SIZE), jnp.float32),
            pltpu.VMEM((SIZE, SIZE), jnp.float32),
            pltpu.VMEM((SIZE,), jnp.float32),
            pltpu.VMEM((SIZE, CHUNK), jnp.float32),
            pltpu.VMEM((SIZE, CHUNK), jnp.float32),
            pltpu.VMEM((L,), jnp.float32),
            pltpu.VMEM((L,), jnp.int32),
            pltpu.SemaphoreType.DMA,
            pltpu.SemaphoreType.DMA,
        ],
    )
    def sc_scan(q_hbm, w_hbm, b_hbm, ckt_hbm, stats_hbm, idx_hbm,
                qv, wv, bv, buf0, buf1, statv, idxv, sem0, sem1):
        wid = lax.axis_index("s") * NC + lax.axis_index("c")
        start = wid * W_KEYS
        io = lax.iota(jnp.int32, L)

        pltpu.sync_copy(q_hbm, qv)
        pltpu.sync_copy(w_hbm, wv)
        pltpu.sync_copy(b_hbm, bv)

        q0 = qv[0, pl.ds(0, L)]
        q1 = qv[0, pl.ds(L, L)]
        q2 = qv[0, pl.ds(2 * L, L)]
        q3 = qv[0, pl.ds(3 * L, L)]

        def proj_block(blk):
            def body(j16, acc):
                row = blk * L + j16
                w0 = wv[row, pl.ds(0, L)]
                w1 = wv[row, pl.ds(L, L)]
                w2 = wv[row, pl.ds(2 * L, L)]
                w3 = wv[row, pl.ds(3 * L, L)]
                s = _tree(q0 * w0 + q1 * w1 + q2 * w2 + q3 * w3, io, jnp.add)
                return jnp.where(io == j16, s, acc)
            acc = lax.fori_loop(0, L, body, jnp.zeros((L,), jnp.float32))
            return acc + bv[pl.ds(blk * L, L)]

        pq = [proj_block(0), proj_block(1), proj_block(2), proj_block(3)]
        pqn2 = _tree(pq[0] * pq[0] + pq[1] * pq[1]
                     + pq[2] * pq[2] + pq[3] * pq[3], io, jnp.add)

        def pq_bcast(c):
            return _perm(pq[c // L], jnp.full((L,), c % L, jnp.int32))

        statv[...] = jnp.full((L,), _NEG, jnp.float32)
        idxv[...] = jnp.zeros((L,), jnp.int32)

        def chunk_start(i):
            return pl.multiple_of(start + i * CHUNK, 128)

        pltpu.make_async_copy(
            ckt_hbm.at[:, pl.ds(chunk_start(0), CHUNK)], buf0, sem0).start()

        def chunk_body(i, _unused):
            p = lax.rem(i, 2)
            s = chunk_start(i)
            nxt = chunk_start(i + 1)
            more = (i + 1) < NCH

            @pl.when(jnp.logical_and(more, p == 0))
            def _():
                pltpu.make_async_copy(
                    ckt_hbm.at[:, pl.ds(nxt, CHUNK)], buf1, sem1).start()

            @pl.when(jnp.logical_and(more, p == 1))
            def _():
                pltpu.make_async_copy(
                    ckt_hbm.at[:, pl.ds(nxt, CHUNK)], buf0, sem0).start()

            def process(buf):
                def group_body(g, car):
                    gbk, gbi = car
                    rb = g * L
                    da = [jnp.zeros((L,), jnp.float32) for _ in range(4)]
                    na = [jnp.zeros((L,), jnp.float32) for _ in range(4)]
                    for c in range(SIZE):
                        col = buf[c, pl.ds(rb, L)]
                        da[c % 4] = da[c % 4] + col * pq_bcast(c)
                        na[c % 4] = na[c % 4] + col * col
                    dd = (da[0] + da[1]) + (da[2] + da[3])
                    nn = (na[0] + na[1]) + (na[2] + na[3])
                    kvec = dd * jnp.abs(dd) / jnp.maximum(nn, jnp.float32(_EPS2))
                    ivec = (s + rb) + io
                    upd = kvec > gbk
                    return (jnp.where(upd, kvec, gbk),
                            jnp.where(upd, ivec, gbi))

                bk, bi = lax.fori_loop(0, GROUPS, group_body,
                                       (statv[...], idxv[...]))
                statv[...] = bk
                idxv[...] = bi

            @pl.when(p == 0)
            def _():
                pltpu.make_async_copy(
                    ckt_hbm.at[:, pl.ds(s, CHUNK)], buf0, sem0).wait()
                process(buf0)

            @pl.when(p == 1)
            def _():
                pltpu.make_async_copy(
                    ckt_hbm.at[:, pl.ds(s, CHUNK)], buf1, sem1).wait()
                process(buf1)

            return 0

        lax.fori_loop(0, NCH, chunk_body, 0)
        bk = statv[...]
        bi = idxv[...]

        m = _tree(bk, io, jnp.maximum)
        sel = bk == m
        bidx = _tree(jnp.where(sel, bi, jnp.int32(2147483647)), io, jnp.minimum)

        sv = jnp.zeros((L,), jnp.float32)
        sv = jnp.where(io == 0, m, sv)
        sv = jnp.where(io == 1, pqn2, sv)
        statv[...] = sv
        idxv[...] = bidx
        pltpu.sync_copy(statv, stats_hbm.at[wid])
        pltpu.sync_copy(idxv, idx_hbm.at[wid])

    return sc_scan


_sc_scan = _make_sc_scan()


def _tc_body(pq_ref, ck_ref, ko_ref, io_ref):
    i = pl.program_id(0)
    pqv = pq_ref[...]
    blk = ck_ref[...]
    dot = jax.lax.dot_general(
        pqv, blk, (((1,), (0,)), ((), ())),
        preferred_element_type=jnp.float32)
    n2 = jnp.sum(blk * blk, axis=0, keepdims=True)
    key = dot * jnp.abs(dot) / jnp.maximum(n2, jnp.float32(_EPS2))
    rows = (SC_END + i * TC_BLOCK
            + jax.lax.broadcasted_iota(jnp.int32, (1, TC_BLOCK), 1))
    bm = jnp.max(key)
    bi = jnp.min(jnp.where(key == bm, rows, jnp.int32(2147483647)))
    ko_ref[...] = jnp.full((1, 1, 1), bm, jnp.float32)
    io_ref[...] = jnp.full((1, 1, 1), bi, jnp.int32)


def _tc_scan(pq, ckt):
    return pl.pallas_call(
        _tc_body,
        grid=(TC_NBLK,),
        in_specs=[
            pl.BlockSpec((1, SIZE), lambda i: (0, 0)),
            pl.BlockSpec((SIZE, TC_BLOCK), lambda i: (0, i + SC_END // TC_BLOCK)),
        ],
        out_specs=[
            pl.BlockSpec((1, 1, 1), lambda i: (i, 0, 0)),
            pl.BlockSpec((1, 1, 1), lambda i: (i, 0, 0)),
        ],
        out_shape=[
            jax.ShapeDtypeStruct((TC_NBLK, 1, 1), jnp.float32),
            jax.ShapeDtypeStruct((TC_NBLK, 1, 1), jnp.int32),
        ],
    )(pq, ckt)


def kernel(query, W, b, cache_keys, cache_values):
    ckt = cache_keys.T
    stats, idxs = _sc_scan(query, W, b, ckt)

    pq = query @ W.T + b
    tk, ti = _tc_scan(pq, ckt)

    key32 = stats[:, 0]
    w = jnp.argmax(key32)
    k_sc = key32[w]
    i_sc = idxs[w, 0]

    tk = tk[:, 0, 0]
    ti = ti[:, 0, 0]
    wt = jnp.argmax(tk)
    k_tc = tk[wt]
    i_tc = ti[wt]

    tail = cache_keys[TC_END:]
    tdot = tail @ pq[0]
    tn2 = jnp.sum(tail * tail, axis=1)
    tkey = tdot * jnp.abs(tdot) / jnp.maximum(tn2, jnp.float32(_EPS2))
    wl = jnp.argmax(tkey)
    k_tl = tkey[wl]
    i_tl = (TC_END + wl).astype(jnp.int32)

    k_best = k_sc
    idx = i_sc
    take_tc = k_tc > k_best
    k_best = jnp.where(take_tc, k_tc, k_best)
    idx = jnp.where(take_tc, i_tc, idx)
    take_tl = k_tl > k_best
    k_best = jnp.where(take_tl, k_tl, k_best)
    idx = jnp.where(take_tl, i_tl, idx)

    pqn = jnp.maximum(jnp.sqrt(jnp.sum(pq * pq)), jnp.float32(1e-8))
    conf = jnp.sign(k_best) * jnp.sqrt(jnp.abs(k_best)) / pqn
    cached_value = lax.dynamic_slice_in_dim(cache_values, idx, 1, axis=0)
    return cached_value, conf


# speedup vs baseline: 4.9945x; 4.9945x over previous
import functools

import jax
import jax.numpy as jnp
from jax import lax
from jax.experimental import pallas as pl
from jax.experimental.pallas import tpu as pltpu
from jax.experimental.pallas import tpu_sc as plsc

SIZE = 64
CAPACITY = 1000000

_info = plsc.get_sparse_core_info()
NC = _info.num_cores
NS = _info.num_subcores
NW = NC * NS
L = 16

CHUNK = 256
W_KEYS = 4864
NCH = W_KEYS // CHUNK
GROUPS = CHUNK // L
SC_END = W_KEYS * NW
TC_BLOCK = 4096
TC_END = 999424
TC_NBLK = (TC_END - SC_END) // TC_BLOCK

_NEG = -3.4e38
_EPS2 = 1e-16


def _perm(v, idx):
    dnums = lax.GatherDimensionNumbers(
        offset_dims=(), collapsed_slice_dims=(0,), start_index_map=(0,))
    return lax.gather(v, idx[:, None], dnums, (1,),
                      mode=lax.GatherScatterMode.PROMISE_IN_BOUNDS)


def _tree(v, io, op):
    for sh in (8, 4, 2, 1):
        v = op(v, _perm(v, io ^ sh))
    return v


def _make_sc_scan():
    mesh = plsc.VectorSubcoreMesh(core_axis_name="c", subcore_axis_name="s")

    @functools.partial(
        pl.kernel,
        out_type=[
            jax.ShapeDtypeStruct((NW, L), jnp.float32),
            jax.ShapeDtypeStruct((NW, L), jnp.int32),
        ],
        mesh=mesh,
        compiler_params=pltpu.CompilerParams(use_tc_tiling_on_sc=True),
        scratch_types=[
            pltpu.VMEM((1, SIZE), jnp.float32),
            pltpu.VMEM((SIZE, SIZE), jnp.float32),
            pltpu.VMEM((SIZE,), jnp.float32),
            pltpu.VMEM((SIZE, CHUNK), jnp.float32),
            pltpu.VMEM((SIZE, CHUNK), jnp.float32),
            pltpu.VMEM((L,), jnp.float32),
            pltpu.VMEM((L,), jnp.int32),
            pltpu.SemaphoreType.DMA,
            pltpu.SemaphoreType.DMA,
        ],
    )
    def sc_scan(q_hbm, w_hbm, b_hbm, ckt_hbm, stats_hbm, idx_hbm,
                qv, wv, bv, buf0, buf1, statv, idxv, sem0, sem1):
        wid = lax.axis_index("s") * NC + lax.axis_index("c")
        start = wid * W_KEYS
        io = lax.iota(jnp.int32, L)

        pltpu.sync_copy(q_hbm, qv)
        pltpu.sync_copy(w_hbm, wv)
        pltpu.sync_copy(b_hbm, bv)

        q0 = qv[0, pl.ds(0, L)]
        q1 = qv[0, pl.ds(L, L)]
        q2 = qv[0, pl.ds(2 * L, L)]
        q3 = qv[0, pl.ds(3 * L, L)]

        def proj_block(blk):
            def body(j16, acc):
                row = blk * L + j16
                w0 = wv[row, pl.ds(0, L)]
                w1 = wv[row, pl.ds(L, L)]
                w2 = wv[row, pl.ds(2 * L, L)]
                w3 = wv[row, pl.ds(3 * L, L)]
                s = _tree(q0 * w0 + q1 * w1 + q2 * w2 + q3 * w3, io, jnp.add)
                return jnp.where(io == j16, s, acc)
            acc = lax.fori_loop(0, L, body, jnp.zeros((L,), jnp.float32))
            return acc + bv[pl.ds(blk * L, L)]

        pq = [proj_block(0), proj_block(1), proj_block(2), proj_block(3)]
        pqn2 = _tree(pq[0] * pq[0] + pq[1] * pq[1]
                     + pq[2] * pq[2] + pq[3] * pq[3], io, jnp.add)

        def pq_bcast(c):
            return _perm(pq[c // L], jnp.full((L,), c % L, jnp.int32))

        statv[...] = jnp.full((L,), _NEG, jnp.float32)
        idxv[...] = jnp.zeros((L,), jnp.int32)

        def chunk_start(i):
            return pl.multiple_of(start + i * CHUNK, 128)

        pltpu.make_async_copy(
            ckt_hbm.at[:, pl.ds(chunk_start(0), CHUNK)], buf0, sem0).start()

        def chunk_body(i, _unused):
            p = lax.rem(i, 2)
            s = chunk_start(i)
            nxt = chunk_start(i + 1)
            more = (i + 1) < NCH

            @pl.when(jnp.logical_and(more, p == 0))
            def _():
                pltpu.make_async_copy(
                    ckt_hbm.at[:, pl.ds(nxt, CHUNK)], buf1, sem1).start()

            @pl.when(jnp.logical_and(more, p == 1))
            def _():
                pltpu.make_async_copy(
                    ckt_hbm.at[:, pl.ds(nxt, CHUNK)], buf0, sem0).start()

            def process(buf):
                def group_body(g, car):
                    gbk, gbi = car
                    rb = g * L
                    da = [jnp.zeros((L,), jnp.float32) for _ in range(4)]
                    na = [jnp.zeros((L,), jnp.float32) for _ in range(4)]
                    for c in range(SIZE):
                        col = buf[c, pl.ds(rb, L)]
                        da[c % 4] = da[c % 4] + col * pq_bcast(c)
                        na[c % 4] = na[c % 4] + col * col
                    dd = (da[0] + da[1]) + (da[2] + da[3])
                    nn = (na[0] + na[1]) + (na[2] + na[3])
                    kvec = dd * jnp.abs(dd) / jnp.maximum(nn, jnp.float32(_EPS2))
                    ivec = (s + rb) + io
                    upd = kvec > gbk
                    return (jnp.where(upd, kvec, gbk),
                            jnp.where(upd, ivec, gbi))

                bk, bi = lax.fori_loop(0, GROUPS, group_body,
                                       (statv[...], idxv[...]))
                statv[...] = bk
                idxv[...] = bi

            @pl.when(p == 0)
            def _():
                pltpu.make_async_copy(
                    ckt_hbm.at[:, pl.ds(s, CHUNK)], buf0, sem0).wait()
                process(buf0)

            @pl.when(p == 1)
            def _():
                pltpu.make_async_copy(
                    ckt_hbm.at[:, pl.ds(s, CHUNK)], buf1, sem1).wait()
                process(buf1)

            return 0

        lax.fori_loop(0, NCH, chunk_body, 0)
        bk = statv[...]
        bi = idxv[...]

        m = _tree(bk, io, jnp.maximum)
        sel = bk == m
        bidx = _tree(jnp.where(sel, bi, jnp.int32(2147483647)), io, jnp.minimum)

        sv = jnp.zeros((L,), jnp.float32)
        sv = jnp.where(io == 0, m, sv)
        sv = jnp.where(io == 1, pqn2, sv)
        statv[...] = sv
        idxv[...] = bidx
        pltpu.sync_copy(statv, stats_hbm.at[wid])
        pltpu.sync_copy(idxv, idx_hbm.at[wid])

    return sc_scan


_sc_scan = _make_sc_scan()


def _tc_body(pq_ref, ck_ref, ko_ref, io_ref):
    i = pl.program_id(0)
    pqv = pq_ref[...]
    blk = ck_ref[...]
    dot = jax.lax.dot_general(
        pqv, blk, (((1,), (0,)), ((), ())),
        preferred_element_type=jnp.float32)
    n2 = jnp.sum(blk * blk, axis=0, keepdims=True)
    key = dot * jnp.abs(dot) / jnp.maximum(n2, jnp.float32(_EPS2))
    rows = (SC_END + i * TC_BLOCK
            + jax.lax.broadcasted_iota(jnp.int32, (1, TC_BLOCK), 1))
    bm = jnp.max(key)
    bi = jnp.min(jnp.where(key == bm, rows, jnp.int32(2147483647)))
    ko_ref[...] = jnp.full((1, 1, 1), bm, jnp.float32)
    io_ref[...] = jnp.full((1, 1, 1), bi, jnp.int32)


def _tc_scan(pq, ckt):
    return pl.pallas_call(
        _tc_body,
        grid=(TC_NBLK,),
        in_specs=[
            pl.BlockSpec((1, SIZE), lambda i: (0, 0)),
            pl.BlockSpec((SIZE, TC_BLOCK), lambda i: (0, i + SC_END // TC_BLOCK)),
        ],
        out_specs=[
            pl.BlockSpec((1, 1, 1), lambda i: (i, 0, 0)),
            pl.BlockSpec((1, 1, 1), lambda i: (i, 0, 0)),
        ],
        out_shape=[
            jax.ShapeDtypeStruct((TC_NBLK, 1, 1), jnp.float32),
            jax.ShapeDtypeStruct((TC_NBLK, 1, 1), jnp.int32),
        ],
    )(pq, ckt)


def kernel(query, W, b, cache_keys, cache_values):
    ckt = cache_keys.T
    stats, idxs = _sc_scan(query, W, b, ckt)

    pq = query @ W.T + b
    tk, ti = _tc_scan(pq, ckt)

    key32 = stats[:, 0]
    w = jnp.argmax(key32)
    k_sc = key32[w]
    i_sc = idxs[w, 0]

    tk = tk[:, 0, 0]
    ti = ti[:, 0, 0]
    wt = jnp.argmax(tk)
    k_tc = tk[wt]
    i_tc = ti[wt]

    tail = cache_keys[TC_END:]
    tdot = tail @ pq[0]
    tn2 = jnp.sum(tail * tail, axis=1)
    tkey = tdot * jnp.abs(tdot) / jnp.maximum(tn2, jnp.float32(_EPS2))
    wl = jnp.argmax(tkey)
    k_tl = tkey[wl]
    i_tl = (TC_END + wl).astype(jnp.int32)

    k_best = k_sc
    idx = i_sc
    take_tc = k_tc > k_best
    k_best = jnp.where(take_tc, k_tc, k_best)
    idx = jnp.where(take_tc, i_tc, idx)
    take_tl = k_tl > k_best
    k_best = jnp.where(take_tl, k_tl, k_best)
    idx = jnp.where(take_tl, i_tl, idx)

    pqn = jnp.maximum(jnp.sqrt(jnp.sum(pq * pq)), jnp.float32(1e-8))
    conf = jnp.sign(k_best) * jnp.sqrt(jnp.abs(k_best)) / pqn
    cached_value = lax.dynamic_slice_in_dim(cache_values, idx, 1, axis=0)
    return cached_value, conf


# speedup vs baseline: 6.9056x; 1.3826x over previous
import functools

import jax
import jax.numpy as jnp
from jax import lax
from jax.experimental import pallas as pl
from jax.experimental.pallas import tpu as pltpu
from jax.experimental.pallas import tpu_sc as plsc

SIZE = 64
CAPACITY = 1000000

_info = plsc.get_sparse_core_info()
NC = _info.num_cores
NS = _info.num_subcores
NW = NC * NS
L = 16

CHUNK = 256
W_KEYS = 4096
NCH = W_KEYS // CHUNK
GROUPS = CHUNK // L
SC_END = W_KEYS * NW
TC_BLOCK = 8192
TC_END = 999424
TC_NBLK = (TC_END - SC_END) // TC_BLOCK

_NEG = -3.4e38
_EPS2 = 1e-16


def _perm(v, idx):
    dnums = lax.GatherDimensionNumbers(
        offset_dims=(), collapsed_slice_dims=(0,), start_index_map=(0,))
    return lax.gather(v, idx[:, None], dnums, (1,),
                      mode=lax.GatherScatterMode.PROMISE_IN_BOUNDS)


def _tree(v, io, op):
    for sh in (8, 4, 2, 1):
        v = op(v, _perm(v, io ^ sh))
    return v


def _make_sc_scan():
    mesh = plsc.VectorSubcoreMesh(core_axis_name="c", subcore_axis_name="s")

    @functools.partial(
        pl.kernel,
        out_type=[
            jax.ShapeDtypeStruct((NW, L), jnp.float32),
            jax.ShapeDtypeStruct((NW, L), jnp.int32),
        ],
        mesh=mesh,
        compiler_params=pltpu.CompilerParams(use_tc_tiling_on_sc=True),
        scratch_types=[
            pltpu.VMEM((1, SIZE), jnp.float32),
            pltpu.VMEM((SIZE, SIZE), jnp.float32),
            pltpu.VMEM((SIZE,), jnp.float32),
            pltpu.VMEM((SIZE, CHUNK), jnp.float32),
            pltpu.VMEM((SIZE, CHUNK), jnp.float32),
            pltpu.VMEM((L,), jnp.float32),
            pltpu.VMEM((L,), jnp.int32),
            pltpu.SemaphoreType.DMA,
            pltpu.SemaphoreType.DMA,
        ],
    )
    def sc_scan(q_hbm, w_hbm, b_hbm, ckt_hbm, stats_hbm, idx_hbm,
                qv, wv, bv, buf0, buf1, statv, idxv, sem0, sem1):
        wid = lax.axis_index("s") * NC + lax.axis_index("c")
        start = wid * W_KEYS
        io = lax.iota(jnp.int32, L)

        pltpu.sync_copy(q_hbm, qv)
        pltpu.sync_copy(w_hbm, wv)
        pltpu.sync_copy(b_hbm, bv)

        q0 = qv[0, pl.ds(0, L)]
        q1 = qv[0, pl.ds(L, L)]
        q2 = qv[0, pl.ds(2 * L, L)]
        q3 = qv[0, pl.ds(3 * L, L)]

        def proj_block(blk):
            def body(j16, acc):
                row = blk * L + j16
                w0 = wv[row, pl.ds(0, L)]
                w1 = wv[row, pl.ds(L, L)]
                w2 = wv[row, pl.ds(2 * L, L)]
                w3 = wv[row, pl.ds(3 * L, L)]
                s = _tree(q0 * w0 + q1 * w1 + q2 * w2 + q3 * w3, io, jnp.add)
                return jnp.where(io == j16, s, acc)
            acc = lax.fori_loop(0, L, body, jnp.zeros((L,), jnp.float32))
            return acc + bv[pl.ds(blk * L, L)]

        pq = [proj_block(0), proj_block(1), proj_block(2), proj_block(3)]
        pqn2 = _tree(pq[0] * pq[0] + pq[1] * pq[1]
                     + pq[2] * pq[2] + pq[3] * pq[3], io, jnp.add)

        def pq_bcast(c):
            return _perm(pq[c // L], jnp.full((L,), c % L, jnp.int32))

        statv[...] = jnp.full((L,), _NEG, jnp.float32)
        idxv[...] = jnp.zeros((L,), jnp.int32)

        def chunk_start(i):
            return pl.multiple_of(start + i * CHUNK, 128)

        pltpu.make_async_copy(
            ckt_hbm.at[:, pl.ds(chunk_start(0), CHUNK)], buf0, sem0).start()

        def chunk_body(i, _unused):
            p = lax.rem(i, 2)
            s = chunk_start(i)
            nxt = chunk_start(i + 1)
            more = (i + 1) < NCH

            @pl.when(jnp.logical_and(more, p == 0))
            def _():
                pltpu.make_async_copy(
                    ckt_hbm.at[:, pl.ds(nxt, CHUNK)], buf1, sem1).start()

            @pl.when(jnp.logical_and(more, p == 1))
            def _():
                pltpu.make_async_copy(
                    ckt_hbm.at[:, pl.ds(nxt, CHUNK)], buf0, sem0).start()

            def process(buf):
                def group_body(g, car):
                    gbk, gbi = car
                    rb = g * L
                    da = [jnp.zeros((L,), jnp.float32) for _ in range(4)]
                    na = [jnp.zeros((L,), jnp.float32) for _ in range(4)]
                    for c in range(SIZE):
                        col = buf[c, pl.ds(rb, L)]
                        da[c % 4] = da[c % 4] + col * pq_bcast(c)
                        na[c % 4] = na[c % 4] + col * col
                    dd = (da[0] + da[1]) + (da[2] + da[3])
                    nn = (na[0] + na[1]) + (na[2] + na[3])
                    kvec = dd * jnp.abs(dd) / jnp.maximum(nn, jnp.float32(_EPS2))
                    ivec = (s + rb) + io
                    upd = kvec > gbk
                    return (jnp.where(upd, kvec, gbk),
                            jnp.where(upd, ivec, gbi))

                bk, bi = lax.fori_loop(0, GROUPS, group_body,
                                       (statv[...], idxv[...]))
                statv[...] = bk
                idxv[...] = bi

            @pl.when(p == 0)
            def _():
                pltpu.make_async_copy(
                    ckt_hbm.at[:, pl.ds(s, CHUNK)], buf0, sem0).wait()
                process(buf0)

            @pl.when(p == 1)
            def _():
                pltpu.make_async_copy(
                    ckt_hbm.at[:, pl.ds(s, CHUNK)], buf1, sem1).wait()
                process(buf1)

            return 0

        lax.fori_loop(0, NCH, chunk_body, 0)
        bk = statv[...]
        bi = idxv[...]

        m = _tree(bk, io, jnp.maximum)
        sel = bk == m
        bidx = _tree(jnp.where(sel, bi, jnp.int32(2147483647)), io, jnp.minimum)

        sv = jnp.zeros((L,), jnp.float32)
        sv = jnp.where(io == 0, m, sv)
        sv = jnp.where(io == 1, pqn2, sv)
        statv[...] = sv
        idxv[...] = bidx
        pltpu.sync_copy(statv, stats_hbm.at[wid])
        pltpu.sync_copy(idxv, idx_hbm.at[wid])

    return sc_scan


_sc_scan = _make_sc_scan()


def _tc_body(pq_ref, ck_ref, ko_ref, io_ref):
    i = pl.program_id(0)
    pqv = pq_ref[...]
    blk = ck_ref[...]
    dot = jax.lax.dot_general(
        pqv, blk, (((1,), (0,)), ((), ())),
        preferred_element_type=jnp.float32)
    n2 = jax.lax.dot_general(
        jnp.ones((1, SIZE), jnp.float32), blk * blk, (((1,), (0,)), ((), ())),
        preferred_element_type=jnp.float32)
    key = dot * jnp.abs(dot) / jnp.maximum(n2, jnp.float32(_EPS2))
    rows = (SC_END + i * TC_BLOCK
            + jax.lax.broadcasted_iota(jnp.int32, (1, TC_BLOCK), 1))
    bm = jnp.max(key)
    bi = jnp.min(jnp.where(key == bm, rows, jnp.int32(2147483647)))
    ko_ref[...] = jnp.full((1, 1, 1), bm, jnp.float32)
    io_ref[...] = jnp.full((1, 1, 1), bi, jnp.int32)


def _tc_scan(pq, ckt):
    return pl.pallas_call(
        _tc_body,
        grid=(TC_NBLK,),
        in_specs=[
            pl.BlockSpec((1, SIZE), lambda i: (0, 0)),
            pl.BlockSpec((SIZE, TC_BLOCK), lambda i: (0, i + SC_END // TC_BLOCK)),
        ],
        out_specs=[
            pl.BlockSpec((1, 1, 1), lambda i: (i, 0, 0)),
            pl.BlockSpec((1, 1, 1), lambda i: (i, 0, 0)),
        ],
        out_shape=[
            jax.ShapeDtypeStruct((TC_NBLK, 1, 1), jnp.float32),
            jax.ShapeDtypeStruct((TC_NBLK, 1, 1), jnp.int32),
        ],
    )(pq, ckt)


def kernel(query, W, b, cache_keys, cache_values):
    ckt = cache_keys.T
    stats, idxs = _sc_scan(query, W, b, ckt)

    pq = query @ W.T + b
    tk, ti = _tc_scan(pq, ckt)

    key32 = stats[:, 0]
    w = jnp.argmax(key32)
    k_sc = key32[w]
    i_sc = idxs[w, 0]

    tk = tk[:, 0, 0]
    ti = ti[:, 0, 0]
    wt = jnp.argmax(tk)
    k_tc = tk[wt]
    i_tc = ti[wt]

    tail = cache_keys[TC_END:]
    tdot = tail @ pq[0]
    tn2 = jnp.sum(tail * tail, axis=1)
    tkey = tdot * jnp.abs(tdot) / jnp.maximum(tn2, jnp.float32(_EPS2))
    wl = jnp.argmax(tkey)
    k_tl = tkey[wl]
    i_tl = (TC_END + wl).astype(jnp.int32)

    k_best = k_sc
    idx = i_sc
    take_tc = k_tc > k_best
    k_best = jnp.where(take_tc, k_tc, k_best)
    idx = jnp.where(take_tc, i_tc, idx)
    take_tl = k_tl > k_best
    k_best = jnp.where(take_tl, k_tl, k_best)
    idx = jnp.where(take_tl, i_tl, idx)

    pqn = jnp.maximum(jnp.sqrt(jnp.sum(pq * pq)), jnp.float32(1e-8))
    conf = jnp.sign(k_best) * jnp.sqrt(jnp.abs(k_best)) / pqn
    cached_value = lax.dynamic_slice_in_dim(cache_values, idx, 1, axis=0)
    return cached_value, conf
